# baseline (speedup 1.0000x reference)
import functools

import jax
import jax.numpy as jnp
from jax import lax
from jax.experimental import pallas as pl
from jax.experimental.pallas import tpu as pltpu
from jax.experimental.pallas import tpu_sc as plsc

BATCH = 4
SEQ = 2048
D = 1024
LANES = 16

NUM_CORES = 2
NUM_SUBCORES = 16
NUM_WORKERS = NUM_CORES * NUM_SUBCORES
ROWS_PER_WORKER = SEQ // NUM_WORKERS
R = 8
GROUPS = ROWS_PER_WORKER // R
NBUF = 3

_MESH = plsc.VectorSubcoreMesh(core_axis_name="c", subcore_axis_name="s")


@functools.partial(
    pl.kernel,
    out_type=jax.ShapeDtypeStruct((BATCH, SEQ, D), jnp.float32),
    mesh=_MESH,
    scratch_types=[
        pltpu.VMEM((NBUF, BATCH, R, D), jnp.float32),
        pltpu.VMEM((NBUF, R, D), jnp.float32),
        pltpu.SemaphoreType.DMA((NBUF,)),
        pltpu.SemaphoreType.DMA((NBUF,)),
    ],
)
def _pos_add_sc(x_hbm, pos_hbm, out_hbm, x_v, pos_v, in_sem, out_sem):
    wid = lax.axis_index("s") * NUM_CORES + lax.axis_index("c")
    row_base = wid * ROWS_PER_WORKER
    in_sems = [in_sem.at[i] for i in range(NBUF)]
    out_sems = [out_sem.at[i] for i in range(NBUF)]

    def issue_in(g):
        slot = g % NBUF
        r0 = row_base + g * R
        return [
            pltpu.async_copy(
                pos_hbm.at[pl.ds(r0, R)], pos_v.at[slot], in_sems[slot]),
            pltpu.async_copy(
                x_hbm.at[:, pl.ds(r0, R)], x_v.at[slot], in_sems[slot]),
        ]

    def issue_out(g):
        slot = g % NBUF
        r0 = row_base + g * R
        return [
            pltpu.async_copy(
                x_v.at[slot], out_hbm.at[:, pl.ds(r0, R)],
                out_sems[slot])
        ]

    def compute(slot):
        @plsc.parallel_loop(0, D, step=LANES, unroll=1)
        def _(off):
            for r in range(R):
                pv = pos_v[slot, r, pl.ds(off, LANES)]
                for b in range(BATCH):
                    plsc.addupdate(x_v.at[slot, b, r, pl.ds(off, LANES)],
                                   pv)

    pending_in = {g: issue_in(g) for g in range(min(NBUF, GROUPS))}
    pending_out = {}
    for g in range(GROUPS):
        nxt = g + NBUF - 1
        if g > 0 and nxt < GROUPS:
            for c in pending_out.pop(g - 1):
                c.wait()
            pending_in[nxt] = issue_in(nxt)
        for c in pending_in.pop(g):
            c.wait()
        compute(g % NBUF)
        pending_out[g] = issue_out(g)
    for g in sorted(pending_out):
        for c in pending_out[g]:
            c.wait()


def kernel(x, pos_table):
    return _pos_add_sc(x, pos_table[:x.shape[1]])


# speedup vs baseline: 1.0507x; 1.0507x over previous
import functools

import jax
import jax.numpy as jnp
from jax import lax
from jax.experimental import pallas as pl
from jax.experimental.pallas import tpu as pltpu
from jax.experimental.pallas import tpu_sc as plsc

BATCH = 4
SEQ = 2048
D = 1024
LANES = 16

NUM_CORES = 2
NUM_SUBCORES = 16
NUM_WORKERS = NUM_CORES * NUM_SUBCORES
ROWS_PER_WORKER = SEQ // NUM_WORKERS
R = 4
GROUPS = ROWS_PER_WORKER // R
NBUF = 6

_MESH = plsc.VectorSubcoreMesh(core_axis_name="c", subcore_axis_name="s")


@functools.partial(
    pl.kernel,
    out_type=jax.ShapeDtypeStruct((BATCH, SEQ, D), jnp.float32),
    mesh=_MESH,
    scratch_types=[
        pltpu.VMEM((NBUF, BATCH, R, D), jnp.float32),
        pltpu.VMEM((NBUF, R, D), jnp.float32),
        pltpu.SemaphoreType.DMA((NBUF,)),
        pltpu.SemaphoreType.DMA((NBUF,)),
    ],
)
def _pos_add_sc(x_hbm, pos_hbm, out_hbm, x_v, pos_v, in_sem, out_sem):
    wid = lax.axis_index("s") * NUM_CORES + lax.axis_index("c")
    row_base = wid * ROWS_PER_WORKER
    in_sems = [in_sem.at[i] for i in range(NBUF)]
    out_sems = [out_sem.at[i] for i in range(NBUF)]

    def issue_in(g):
        slot = g % NBUF
        r0 = row_base + g * R
        return [
            pltpu.async_copy(
                pos_hbm.at[pl.ds(r0, R)], pos_v.at[slot], in_sems[slot]),
            pltpu.async_copy(
                x_hbm.at[:, pl.ds(r0, R)], x_v.at[slot], in_sems[slot]),
        ]

    def issue_out(g):
        slot = g % NBUF
        r0 = row_base + g * R
        return [
            pltpu.async_copy(
                x_v.at[slot], out_hbm.at[:, pl.ds(r0, R)],
                out_sems[slot])
        ]

    def compute(slot):
        @plsc.parallel_loop(0, D, step=LANES, unroll=1)
        def _(off):
            for r in range(R):
                pv = pos_v[slot, r, pl.ds(off, LANES)]
                for b in range(BATCH):
                    plsc.addupdate(x_v.at[slot, b, r, pl.ds(off, LANES)],
                                   pv)

    pending_in = {g: issue_in(g) for g in range(min(NBUF, GROUPS))}
    pending_out = {}
    for g in range(GROUPS):
        nxt = g + NBUF - 1
        if g > 0 and nxt < GROUPS:
            for c in pending_out.pop(g - 1):
                c.wait()
            pending_in[nxt] = issue_in(nxt)
        for c in pending_in.pop(g):
            c.wait()
        compute(g % NBUF)
        pending_out[g] = issue_out(g)
    for g in sorted(pending_out):
        for c in pending_out[g]:
            c.wait()


def kernel(x, pos_table):
    return _pos_add_sc(x, pos_table[:x.shape[1]])
